# Initial kernel scaffold; baseline (speedup 1.0000x reference)
import jax
import jax.numpy as jnp
from jax import lax
from jax.experimental import pallas as pl
from jax.experimental.pallas import tpu as pltpu
from jax.experimental.pallas import tpu_sc as plsc

N_NODES = 10000
N_EDGES = 320000
DIM = 128

NC = 2
NS = 16
NW = NC * NS
EPW = N_EDGES // NW
CHUNK = 80
NCHUNK = EPW // CHUNK
ROWS_PER_TILE = N_NODES // NS
DEN_PAD = 10240


def _matmul_body(x_ref, wfc_ref, wa_ref, z_ref, st_ref):
    xb = x_ref[...]
    zb = lax.dot_general(xb, wfc_ref[...], (((1,), (1,)), ((), ())),
                         preferred_element_type=jnp.float32)
    z_ref[...] = zb
    st_ref[...] = lax.dot_general(wa_ref[...], zb, (((1,), (1,)), ((), ())),
                                  preferred_element_type=jnp.float32)


def _combine_body(acc_ref, den_ref, out_ref):
    a = acc_ref[0] + acc_ref[1]
    d = den_ref[0] + den_ref[1]
    d = jnp.where(d == 0.0, 1.0, d)
    out_ref[...] = a / d[:, None]


def _sc_body(st_hbm, src_hbm, dst_hbm, z_hbm, acc_out, den_out,
             s_v, t_v, src_v, dst_v, rows_v, w_v, zb_v, zd_v,
             acc_sp, den_sp):
    c = lax.axis_index("c")
    s = lax.axis_index("s")
    wid = c * NS + s

    zero16 = jnp.zeros((16,), jnp.float32)

    def _zrow(r, carry):
        for q in range(DIM // 16):
            zb_v[r, pl.ds(q * 16, 16)] = zero16
        return carry
    lax.fori_loop(0, 125, _zrow, 0)

    def _zden(r, carry):
        zd_v[pl.ds(r * 16, 16)] = zero16
        return carry
    lax.fori_loop(0, 40, _zden, 0)

    for i in range(5):
        pltpu.sync_copy(zb_v, acc_sp.at[pl.ds((s * 5 + i) * 125, 125)])
    pltpu.sync_copy(zd_v, den_sp.at[pl.ds(s * 640, 640)])

    pltpu.sync_copy(st_hbm.at[0], s_v)
    pltpu.sync_copy(st_hbm.at[1], t_v)
    pltpu.sync_copy(src_hbm.at[wid], src_v)
    pltpu.sync_copy(dst_hbm.at[wid], dst_v)

    plsc.subcore_barrier()

    def _chunk(j, carry):
        pltpu.sync_copy(z_hbm.at[src_v.at[j]], rows_v)
        for k in range(CHUNK // 16):
            si = src_v[j, pl.ds(k * 16, 16)]
            di = dst_v[j, pl.ds(k * 16, 16)]
            sv = plsc.load_gather(s_v, [si])
            tv = plsc.load_gather(t_v, [di])
            e = sv + tv
            e = jnp.where(e >= 0.0, e, 0.01 * e)
            w_v[pl.ds(k * 16, 16)] = jnp.exp(e)
        def _srow(b, carry2):
            wb = w_v[b]
            for q in range(DIM // 16):
                rows_v[b, pl.ds(q * 16, 16)] = rows_v[b, pl.ds(q * 16, 16)] * wb
            return carry2
        lax.fori_loop(0, CHUNK, _srow, 0)
        pltpu.sync_copy(rows_v, acc_sp.at[dst_v.at[j]], add=True)
        pltpu.sync_copy(w_v, den_sp.at[dst_v.at[j]], add=True)
        return carry
    lax.fori_loop(0, NCHUNK, _chunk, 0)

    plsc.subcore_barrier()

    pltpu.sync_copy(acc_sp.at[pl.ds(s * ROWS_PER_TILE, ROWS_PER_TILE)],
                    acc_out.at[c, pl.ds(s * ROWS_PER_TILE, ROWS_PER_TILE)])
    pltpu.sync_copy(den_sp.at[pl.ds(s * 640, 640)], den_out.at[c, s])


@jax.jit
def kernel(x, edge_index, W_fc, W_attn):
    x = x.astype(jnp.float32)
    W_fc = W_fc.astype(jnp.float32)
    W_attn = W_attn.astype(jnp.float32)
    wa = jnp.zeros((8, DIM), jnp.float32)
    wa = wa.at[0].set(W_attn[0, :DIM]).at[1].set(W_attn[0, DIM:])

    src = edge_index[0].astype(jnp.int32).reshape(NW, NCHUNK, CHUNK)
    dst = edge_index[1].astype(jnp.int32).reshape(NW, NCHUNK, CHUNK)

    bn = 2000
    z, st = pl.pallas_call(
        _matmul_body,
        grid=(N_NODES // bn,),
        in_specs=[
            pl.BlockSpec((bn, DIM), lambda i: (i, 0)),
            pl.BlockSpec((DIM, DIM), lambda i: (0, 0)),
            pl.BlockSpec((8, DIM), lambda i: (0, 0)),
        ],
        out_specs=[
            pl.BlockSpec((bn, DIM), lambda i: (i, 0)),
            pl.BlockSpec((8, bn), lambda i: (0, i)),
        ],
        out_shape=[
            jax.ShapeDtypeStruct((N_NODES, DIM), jnp.float32),
            jax.ShapeDtypeStruct((8, N_NODES), jnp.float32),
        ],
    )(x, W_fc, wa)

    sc = pl.kernel(
        _sc_body,
        out_type=[
            jax.ShapeDtypeStruct((NC, N_NODES, DIM), jnp.float32),
            jax.ShapeDtypeStruct((NC, NS, 640), jnp.float32),
        ],
        mesh=plsc.VectorSubcoreMesh(core_axis_name="c", subcore_axis_name="s"),
        scratch_types=[
            pltpu.VMEM((N_NODES,), jnp.float32),
            pltpu.VMEM((N_NODES,), jnp.float32),
            pltpu.VMEM((NCHUNK, CHUNK), jnp.int32),
            pltpu.VMEM((NCHUNK, CHUNK), jnp.int32),
            pltpu.VMEM((CHUNK, DIM), jnp.float32),
            pltpu.VMEM((CHUNK,), jnp.float32),
            pltpu.VMEM((125, DIM), jnp.float32),
            pltpu.VMEM((640,), jnp.float32),
            pltpu.VMEM_SHARED((N_NODES, DIM), jnp.float32),
            pltpu.VMEM_SHARED((DEN_PAD,), jnp.float32),
        ],
    )
    acc, den = sc(st, src, dst, z)

    den = den.reshape(NC, DEN_PAD)[:, :N_NODES]

    h = pl.pallas_call(
        _combine_body,
        grid=(N_NODES // bn,),
        in_specs=[
            pl.BlockSpec((NC, bn, DIM), lambda i: (0, i, 0)),
            pl.BlockSpec((NC, bn), lambda i: (0, i)),
        ],
        out_specs=pl.BlockSpec((bn, DIM), lambda i: (i, 0)),
        out_shape=jax.ShapeDtypeStruct((N_NODES, DIM), jnp.float32),
    )(acc, den)
    return h


# speedup vs baseline: 22.3775x; 22.3775x over previous
import jax
import jax.numpy as jnp
from jax import lax
from jax.experimental import pallas as pl
from jax.experimental.pallas import tpu as pltpu
from jax.experimental.pallas import tpu_sc as plsc

N_NODES = 10000
N_EDGES = 320000
DIM = 128

NC = 2
NS = 16
NW = NC * NS
CHUNK = 128
NCHUNK = 79
EPAD = NW * NCHUNK * CHUNK - N_EDGES
NPAD = 10240
RPT = NPAD // NS


def _matmul_body(x_ref, wfc_ref, wa_ref, z_ref, st_ref):
    xb = x_ref[...]
    zb = lax.dot_general(xb, wfc_ref[...], (((1,), (1,)), ((), ())),
                         preferred_element_type=jnp.float32)
    z_ref[...] = zb
    st_ref[...] = lax.dot_general(wa_ref[...], zb, (((1,), (1,)), ((), ())),
                                  preferred_element_type=jnp.float32)


def _combine_body(acc_ref, den_ref, out_ref):
    a = acc_ref[0, :N_NODES] + acc_ref[1, :N_NODES]
    d = den_ref[0, :N_NODES] + den_ref[1, :N_NODES]
    d = jnp.where(d == 0.0, 1.0, d)
    out_ref[...] = a / d[:, None]


def _sc_body(st_hbm, src_hbm, dst_hbm, z_hbm, acc_out, den_out,
             src_v, dst_v, rows_v, w_v, sch_v, tch_v,
             s_sp, t_sp, acc_sp, den_sp):
    c = lax.axis_index("c")
    s = lax.axis_index("s")
    wid = c * NS + s

    zero16 = jnp.zeros((16,), jnp.float32)

    def _zrow(r, carry):
        for q in range(DIM // 16):
            rows_v[r, pl.ds(q * 16, 16)] = zero16
        return carry
    lax.fori_loop(0, CHUNK, _zrow, 0)

    for i in range(RPT // CHUNK):
        pltpu.sync_copy(rows_v, acc_sp.at[pl.ds(s * RPT + i * CHUNK, CHUNK)])
    for i in range(RPT // CHUNK):
        pltpu.sync_copy(rows_v.at[0], den_sp.at[pl.ds(s * RPT + i * CHUNK, CHUNK)])

    @pl.when(s == 0)
    def _stage():
        pltpu.sync_copy(st_hbm.at[0], s_sp)
        pltpu.sync_copy(st_hbm.at[1], t_sp)
    pltpu.sync_copy(src_hbm.at[wid], src_v)
    pltpu.sync_copy(dst_hbm.at[wid], dst_v)

    plsc.subcore_barrier()

    def _chunk(j, carry):
        pltpu.sync_copy(z_hbm.at[src_v.at[j]], rows_v)
        pltpu.sync_copy(s_sp.at[src_v.at[j]], sch_v)
        pltpu.sync_copy(t_sp.at[dst_v.at[j]], tch_v)
        for k in range(CHUNK // 16):
            e = sch_v[pl.ds(k * 16, 16)] + tch_v[pl.ds(k * 16, 16)]
            e = jnp.where(e >= 0.0, e, 0.01 * e)
            w_v[pl.ds(k * 16, 16)] = jnp.exp(e)
        def _srow(b, carry2):
            wb = w_v[pl.ds(b, 16)][0]
            for q in range(DIM // 16):
                rows_v[b, pl.ds(q * 16, 16)] = rows_v[b, pl.ds(q * 16, 16)] * wb
            return carry2
        lax.fori_loop(0, CHUNK, _srow, 0)
        pltpu.sync_copy(rows_v, acc_sp.at[dst_v.at[j]], add=True)
        pltpu.sync_copy(w_v.at[pl.ds(0, CHUNK)], den_sp.at[dst_v.at[j]], add=True)
        return carry
    lax.fori_loop(0, NCHUNK, _chunk, 0)

    plsc.subcore_barrier()

    pltpu.sync_copy(acc_sp.at[pl.ds(s * RPT, RPT)],
                    acc_out.at[c, pl.ds(s * RPT, RPT)])
    pltpu.sync_copy(den_sp.at[pl.ds(s * RPT, RPT)],
                    den_out.at[pl.ds(c * NPAD + s * RPT, RPT)])


@jax.jit
def kernel(x, edge_index, W_fc, W_attn):
    x = x.astype(jnp.float32)
    W_fc = W_fc.astype(jnp.float32)
    W_attn = W_attn.astype(jnp.float32)
    wa = jnp.zeros((8, DIM), jnp.float32)
    wa = wa.at[0].set(W_attn[0, :DIM]).at[1].set(W_attn[0, DIM:])

    pad_src = (jnp.arange(EPAD, dtype=jnp.int32) * 7) % N_NODES
    pad_dst = N_NODES + (jnp.arange(EPAD, dtype=jnp.int32) % (NPAD - N_NODES))
    src = jnp.concatenate([edge_index[0].astype(jnp.int32), pad_src])
    dst = jnp.concatenate([edge_index[1].astype(jnp.int32), pad_dst])
    src = src.reshape(NW, NCHUNK, CHUNK)
    dst = dst.reshape(NW, NCHUNK, CHUNK)

    z, st = pl.pallas_call(
        _matmul_body,
        out_shape=[
            jax.ShapeDtypeStruct((N_NODES, DIM), jnp.float32),
            jax.ShapeDtypeStruct((8, N_NODES), jnp.float32),
        ],
    )(x, W_fc, wa)

    sc = pl.kernel(
        _sc_body,
        out_type=[
            jax.ShapeDtypeStruct((NC, NPAD, DIM), jnp.float32),
            jax.ShapeDtypeStruct((NC * NPAD,), jnp.float32),
        ],
        mesh=plsc.VectorSubcoreMesh(core_axis_name="c", subcore_axis_name="s"),
        compiler_params=pltpu.CompilerParams(needs_layout_passes=False),
        scratch_types=[
            pltpu.VMEM((NCHUNK, CHUNK), jnp.int32),
            pltpu.VMEM((NCHUNK, CHUNK), jnp.int32),
            pltpu.VMEM((CHUNK, DIM), jnp.float32),
            pltpu.VMEM((CHUNK + 16,), jnp.float32),
            pltpu.VMEM((CHUNK,), jnp.float32),
            pltpu.VMEM((CHUNK,), jnp.float32),
            pltpu.VMEM_SHARED((N_NODES,), jnp.float32),
            pltpu.VMEM_SHARED((N_NODES,), jnp.float32),
            pltpu.VMEM_SHARED((NPAD, DIM), jnp.float32),
            pltpu.VMEM_SHARED((NPAD,), jnp.float32),
        ],
    )
    acc, den = sc(st, src, dst, z)

    den = den.reshape(NC, NPAD)

    h = pl.pallas_call(
        _combine_body,
        out_shape=jax.ShapeDtypeStruct((N_NODES, DIM), jnp.float32),
    )(acc, den)
    return h
